# baseline (device time: 38359 ns/iter reference)
import jax
import jax.numpy as jnp
from jax import lax
from jax.experimental import pallas as pl
from jax.experimental.pallas import tpu as pltpu

M_PER = 2048
N_OUT = 512
H = M_PER // 2
C = 8
R = H // C


def kernel(x):
    m, n = x.shape
    assert (m, n) == (M_PER, 2 * N_OUT)

    def body(x_ref, out_ref, stage_ref, ys_sem, yr_sem, xs_sem, xr_sem):
        my_x = lax.axis_index("x")
        my_y = lax.axis_index("y")
        my_z = lax.axis_index("z")
        ox = 1 - my_x
        oy = 1 - my_y
        y_nbr = (my_x, oy, my_z)
        x_nbr = (ox, my_y, my_z)

        barrier_sem = pltpu.get_barrier_semaphore()
        for nbr in (y_nbr, x_nbr):
            pl.semaphore_signal(
                barrier_sem, inc=1,
                device_id=nbr, device_id_type=pl.DeviceIdType.MESH,
            )
        pl.semaphore_wait(barrier_sem, 2)

        src_row0 = my_x * H
        dst_row0 = my_y * M_PER + my_x * H
        fwd_row0 = oy * M_PER + my_x * H
        oth_row0 = oy * M_PER + ox * H

        stage_ref[:, :] = x_ref[
            pl.ds(src_row0, H), pl.ds(oy * N_OUT, N_OUT)
        ]

        y_sends = []
        for c in range(C):
            r = pltpu.make_async_remote_copy(
                src_ref=stage_ref.at[pl.ds(c * R, R), :],
                dst_ref=out_ref.at[pl.ds(dst_row0 + c * R, R), :],
                send_sem=ys_sem.at[c],
                recv_sem=yr_sem.at[c],
                device_id=y_nbr,
                device_id_type=pl.DeviceIdType.MESH,
            )
            r.start()
            y_sends.append(r)

        x_sends = []
        for c in range(C):
            recv_y = pltpu.make_async_remote_copy(
                src_ref=x_ref.at[pl.ds(0, R), pl.ds(0, N_OUT)],
                dst_ref=out_ref.at[pl.ds(fwd_row0 + c * R, R), :],
                send_sem=ys_sem.at[c],
                recv_sem=yr_sem.at[c],
                device_id=y_nbr,
                device_id_type=pl.DeviceIdType.MESH,
            )
            recv_y.wait_recv()
            fwd = pltpu.make_async_remote_copy(
                src_ref=out_ref.at[pl.ds(fwd_row0 + c * R, R), :],
                dst_ref=out_ref.at[pl.ds(fwd_row0 + c * R, R), :],
                send_sem=xs_sem.at[c],
                recv_sem=xr_sem.at[c],
                device_id=x_nbr,
                device_id_type=pl.DeviceIdType.MESH,
            )
            fwd.start()
            x_sends.append(fwd)

        out_ref[pl.ds(my_y * M_PER, M_PER), :] = x_ref[
            :, pl.ds(my_y * N_OUT, N_OUT)
        ]

        for c in range(C):
            recv_x = pltpu.make_async_remote_copy(
                src_ref=x_ref.at[pl.ds(0, R), pl.ds(0, N_OUT)],
                dst_ref=out_ref.at[pl.ds(oth_row0 + c * R, R), :],
                send_sem=xs_sem.at[c],
                recv_sem=xr_sem.at[c],
                device_id=x_nbr,
                device_id_type=pl.DeviceIdType.MESH,
            )
            recv_x.wait_recv()
        for c in range(C):
            y_sends[c].wait_send()
            x_sends[c].wait_send()

    return pl.pallas_call(
        body,
        out_shape=jax.ShapeDtypeStruct((2 * M_PER, N_OUT), x.dtype),
        in_specs=[pl.BlockSpec(memory_space=pltpu.VMEM)],
        out_specs=pl.BlockSpec(memory_space=pltpu.VMEM),
        scratch_shapes=[
            pltpu.VMEM((H, N_OUT), x.dtype),
            pltpu.SemaphoreType.DMA((C,)),
            pltpu.SemaphoreType.DMA((C,)),
            pltpu.SemaphoreType.DMA((C,)),
            pltpu.SemaphoreType.DMA((C,)),
        ],
        compiler_params=pltpu.CompilerParams(collective_id=0),
    )(x)


# device time: 38204 ns/iter; 1.0041x vs baseline; 1.0041x over previous
import jax
import jax.numpy as jnp
from jax import lax
from jax.experimental import pallas as pl
from jax.experimental.pallas import tpu as pltpu

M_PER = 2048
N_OUT = 512
H = M_PER // 2
C = 8
R = H // C


def kernel(x):
    m, n = x.shape
    assert (m, n) == (M_PER, 2 * N_OUT)

    def body(x_ref, out_ref, local_sem, ys_sem, yr_sem, xs_sem, xr_sem):
        my_x = lax.axis_index("x")
        my_y = lax.axis_index("y")
        my_z = lax.axis_index("z")
        ox = 1 - my_x
        oy = 1 - my_y
        y_nbr = (my_x, oy, my_z)
        x_nbr = (ox, my_y, my_z)

        barrier_sem = pltpu.get_barrier_semaphore()
        for nbr in (y_nbr, x_nbr):
            pl.semaphore_signal(
                barrier_sem, inc=1,
                device_id=nbr, device_id_type=pl.DeviceIdType.MESH,
            )
        pl.semaphore_wait(barrier_sem, 2)

        src_row0 = my_x * H
        dst_row0 = my_y * M_PER + my_x * H
        fwd_row0 = oy * M_PER + my_x * H
        oth_row0 = oy * M_PER + ox * H

        y_sends = []
        for c in range(C):
            r = pltpu.make_async_remote_copy(
                src_ref=x_ref.at[
                    pl.ds(src_row0 + c * R, R), pl.ds(oy * N_OUT, N_OUT)
                ],
                dst_ref=out_ref.at[pl.ds(dst_row0 + c * R, R), :],
                send_sem=ys_sem.at[c],
                recv_sem=yr_sem.at[c],
                device_id=y_nbr,
                device_id_type=pl.DeviceIdType.MESH,
            )
            r.start()
            y_sends.append(r)

        local = pltpu.make_async_copy(
            x_ref.at[:, pl.ds(my_y * N_OUT, N_OUT)],
            out_ref.at[pl.ds(my_y * M_PER, M_PER), :],
            local_sem,
        )
        local.start()

        x_sends = []
        for c in range(C):
            recv_y = pltpu.make_async_remote_copy(
                src_ref=x_ref.at[pl.ds(0, R), pl.ds(0, N_OUT)],
                dst_ref=out_ref.at[pl.ds(fwd_row0 + c * R, R), :],
                send_sem=ys_sem.at[c],
                recv_sem=yr_sem.at[c],
                device_id=y_nbr,
                device_id_type=pl.DeviceIdType.MESH,
            )
            recv_y.wait_recv()
            fwd = pltpu.make_async_remote_copy(
                src_ref=out_ref.at[pl.ds(fwd_row0 + c * R, R), :],
                dst_ref=out_ref.at[pl.ds(fwd_row0 + c * R, R), :],
                send_sem=xs_sem.at[c],
                recv_sem=xr_sem.at[c],
                device_id=x_nbr,
                device_id_type=pl.DeviceIdType.MESH,
            )
            fwd.start()
            x_sends.append(fwd)

        for c in range(C):
            recv_x = pltpu.make_async_remote_copy(
                src_ref=x_ref.at[pl.ds(0, R), pl.ds(0, N_OUT)],
                dst_ref=out_ref.at[pl.ds(oth_row0 + c * R, R), :],
                send_sem=xs_sem.at[c],
                recv_sem=xr_sem.at[c],
                device_id=x_nbr,
                device_id_type=pl.DeviceIdType.MESH,
            )
            recv_x.wait_recv()
        local.wait()
        for c in range(C):
            y_sends[c].wait_send()
            x_sends[c].wait_send()

    return pl.pallas_call(
        body,
        out_shape=jax.ShapeDtypeStruct((2 * M_PER, N_OUT), x.dtype),
        in_specs=[pl.BlockSpec(memory_space=pl.ANY)],
        out_specs=pl.BlockSpec(memory_space=pl.ANY),
        scratch_shapes=[
            pltpu.SemaphoreType.DMA,
            pltpu.SemaphoreType.DMA((C,)),
            pltpu.SemaphoreType.DMA((C,)),
            pltpu.SemaphoreType.DMA((C,)),
            pltpu.SemaphoreType.DMA((C,)),
        ],
        compiler_params=pltpu.CompilerParams(collective_id=0),
    )(x)


# device time: 37125 ns/iter; 1.0332x vs baseline; 1.0291x over previous
import jax
import jax.numpy as jnp
from jax import lax
from jax.experimental import pallas as pl
from jax.experimental.pallas import tpu as pltpu

M_PER = 2048
N_OUT = 512
H = M_PER // 2
C = 16
R = H // C


def kernel(x):
    m, n = x.shape
    assert (m, n) == (M_PER, 2 * N_OUT)

    def body(x_ref, out_ref, local_sem, ys_sem, yr_sem, xs_sem, xr_sem):
        my_x = lax.axis_index("x")
        my_y = lax.axis_index("y")
        my_z = lax.axis_index("z")
        ox = 1 - my_x
        oy = 1 - my_y
        y_nbr = (my_x, oy, my_z)
        x_nbr = (ox, my_y, my_z)

        barrier_sem = pltpu.get_barrier_semaphore()
        for nbr in (y_nbr, x_nbr):
            pl.semaphore_signal(
                barrier_sem, inc=1,
                device_id=nbr, device_id_type=pl.DeviceIdType.MESH,
            )
        pl.semaphore_wait(barrier_sem, 2)

        src_row0 = my_x * H
        dst_row0 = my_y * M_PER + my_x * H
        fwd_row0 = oy * M_PER + my_x * H
        oth_row0 = oy * M_PER + ox * H

        y_sends = []
        for c in range(C):
            r = pltpu.make_async_remote_copy(
                src_ref=x_ref.at[
                    pl.ds(src_row0 + c * R, R), pl.ds(oy * N_OUT, N_OUT)
                ],
                dst_ref=out_ref.at[pl.ds(dst_row0 + c * R, R), :],
                send_sem=ys_sem.at[c],
                recv_sem=yr_sem.at[c],
                device_id=y_nbr,
                device_id_type=pl.DeviceIdType.MESH,
            )
            r.start()
            y_sends.append(r)

        local = pltpu.make_async_copy(
            x_ref.at[:, pl.ds(my_y * N_OUT, N_OUT)],
            out_ref.at[pl.ds(my_y * M_PER, M_PER), :],
            local_sem,
        )
        local.start()

        x_sends = []
        for c in range(C):
            recv_y = pltpu.make_async_remote_copy(
                src_ref=x_ref.at[pl.ds(0, R), pl.ds(0, N_OUT)],
                dst_ref=out_ref.at[pl.ds(fwd_row0 + c * R, R), :],
                send_sem=ys_sem.at[c],
                recv_sem=yr_sem.at[c],
                device_id=y_nbr,
                device_id_type=pl.DeviceIdType.MESH,
            )
            recv_y.wait_recv()
            fwd = pltpu.make_async_remote_copy(
                src_ref=out_ref.at[pl.ds(fwd_row0 + c * R, R), :],
                dst_ref=out_ref.at[pl.ds(fwd_row0 + c * R, R), :],
                send_sem=xs_sem.at[c],
                recv_sem=xr_sem.at[c],
                device_id=x_nbr,
                device_id_type=pl.DeviceIdType.MESH,
            )
            fwd.start()
            x_sends.append(fwd)

        for c in range(C):
            recv_x = pltpu.make_async_remote_copy(
                src_ref=x_ref.at[pl.ds(0, R), pl.ds(0, N_OUT)],
                dst_ref=out_ref.at[pl.ds(oth_row0 + c * R, R), :],
                send_sem=xs_sem.at[c],
                recv_sem=xr_sem.at[c],
                device_id=x_nbr,
                device_id_type=pl.DeviceIdType.MESH,
            )
            recv_x.wait_recv()
        local.wait()
        for c in range(C):
            y_sends[c].wait_send()
            x_sends[c].wait_send()

    return pl.pallas_call(
        body,
        out_shape=jax.ShapeDtypeStruct((2 * M_PER, N_OUT), x.dtype),
        in_specs=[pl.BlockSpec(memory_space=pl.ANY)],
        out_specs=pl.BlockSpec(memory_space=pl.ANY),
        scratch_shapes=[
            pltpu.SemaphoreType.DMA,
            pltpu.SemaphoreType.DMA((C,)),
            pltpu.SemaphoreType.DMA((C,)),
            pltpu.SemaphoreType.DMA((C,)),
            pltpu.SemaphoreType.DMA((C,)),
        ],
        compiler_params=pltpu.CompilerParams(collective_id=0),
    )(x)
